# baseline (speedup 1.0000x reference)
import functools

import jax
import jax.numpy as jnp
from jax import lax
from jax.experimental import pallas as pl
from jax.experimental.pallas import tpu as pltpu
from jax.experimental.pallas import tpu_sc as plsc

B = 16384
F = 26
V = 100000
D = 16
N = 13

NC = 2
NS = 16
L = 16
NW = NC * NS
BPW = B // NW
CH = 128

VP = 102400
PHASES = (17, 9)


def _score_body(f_lo, w_ref, t_ref, s_ref):
    fi = pl.program_id(0) + f_lo
    x = t_ref[0]
    wv = w_ref[pl.ds(fi, 1)]
    s_ref[...] = jnp.dot(wv, x)[0]


def _score_tc(t2, w2d, nf, f_lo):
    return pl.pallas_call(
        functools.partial(_score_body, f_lo),
        grid=(nf,),
        compiler_params=pltpu.CompilerParams(
            dimension_semantics=("parallel",)),
        in_specs=[
            pl.BlockSpec((F, D), lambda f: (0, 0)),
            pl.BlockSpec((1, D, VP), lambda f: (f + f_lo, 0, 0)),
        ],
        out_specs=pl.BlockSpec((VP,), lambda f: (f,)),
        out_shape=jax.ShapeDtypeStruct((nf * VP,), jnp.float32),
    )(w2d, t2)


def _detile_body(c_in, n_in, c1d, n1d, cw, cv, sem):
    wid = lax.axis_index("s") * NC + lax.axis_index("c")
    base = wid * BPW
    cps = []
    for f in range(F + N):
        src = (c_in.at[pl.ds(f, 1), pl.ds(base, BPW)] if f < F
               else n_in.at[pl.ds(f - F, 1), pl.ds(base, BPW)])
        cps.append(pltpu.async_copy(src, cw.at[pl.ds(f, 1)], sem))
    for cp in cps:
        cp.wait()
    for f in range(F + N):
        def kbody(k, carry, f=f):
            sl = pl.ds(k * L, L)
            cv[pl.ds(f * BPW + k * L, L)] = cw[f, sl]
            return carry
        lax.fori_loop(0, BPW // L, kbody, 0)
    cps = []
    for f in range(F + N):
        dst = (c1d.at[pl.ds(f * B + base, BPW)] if f < F
               else n1d.at[pl.ds((f - F) * B + base, BPW)])
        cps.append(pltpu.async_copy(cv.at[pl.ds(f * BPW, BPW)], dst, sem))
    for cp in cps:
        cp.wait()


def _detile_sc(c_in, n_in):
    mesh = plsc.VectorSubcoreMesh(core_axis_name="c", subcore_axis_name="s")
    f = pl.kernel(
        _detile_body,
        mesh=mesh,
        compiler_params=pltpu.CompilerParams(use_tc_tiling_on_sc=True),
        out_type=[
            jax.ShapeDtypeStruct((F * B,), jnp.int32),
            jax.ShapeDtypeStruct((N * B,), jnp.int32),
        ],
        scratch_types=[
            pltpu.VMEM((F + N, BPW), jnp.int32),
            pltpu.VMEM(((F + N) * BPW,), jnp.int32),
            pltpu.SemaphoreType.DMA,
        ],
    )
    return f(c_in, n_in)


def _gather_body(nf, f_lo, with_nb, s1d, c1d, n1d, wnb, prev, out,
                 cbuf, nbuf, gbuf, wnbv, outv, pbuf, sem, gsem):
    nst = (nf * BPW) // CH
    wid = lax.axis_index("s") * NC + lax.axis_index("c")
    base = wid * BPW

    cps = [pltpu.async_copy(wnb, wnbv, sem)]
    for fl in range(nf):
        cps.append(pltpu.async_copy(
            c1d.at[pl.ds((f_lo + fl) * B + base, BPW)],
            cbuf.at[pl.ds(fl * BPW, BPW)], sem))
    if with_nb:
        for n in range(N):
            cps.append(pltpu.async_copy(n1d.at[pl.ds(n * B + base, BPW)],
                                        nbuf.at[pl.ds(n * BPW, BPW)], sem))
    else:
        cps.append(pltpu.async_copy(prev.at[pl.ds(base, BPW)], pbuf, sem))
    for cp in cps:
        cp.wait()

    for i in range(nst):
        fl = i // (BPW // CH)
        sl = pl.ds(i * CH, CH)
        pltpu.async_copy(s1d.at[pl.ds(fl * VP, VP)].at[cbuf.at[sl]],
                         gbuf.at[sl], gsem)

    pltpu.make_async_copy(s1d.at[pl.ds(0, nf * BPW)], gbuf, gsem).wait()

    wn_b = [wnbv[n] for n in range(N)]
    bias = wnbv[N]

    def gbody(g, carry):
        val = gbuf[pl.ds(g * L, L)]
        for fl in range(1, nf):
            val = val + gbuf[pl.ds(fl * BPW + g * L, L)]
        if with_nb:
            val = val + bias
            for n in range(N):
                val = val + nbuf[pl.ds(n * BPW + g * L, L)] * wn_b[n]
        else:
            val = val + pbuf[pl.ds(g * L, L)]
        outv[pl.ds(g * L, L)] = val
        return carry
    lax.fori_loop(0, BPW // L, gbody, 0)

    pltpu.sync_copy(outv, out.at[pl.ds(base, BPW)])


def _gather_sc(s1d, c1d, n1d, wnb, prev, nf, f_lo, with_nb):
    mesh = plsc.VectorSubcoreMesh(core_axis_name="c", subcore_axis_name="s")
    f = pl.kernel(
        functools.partial(_gather_body, nf, f_lo, with_nb),
        mesh=mesh,
        compiler_params=pltpu.CompilerParams(use_tc_tiling_on_sc=False),
        out_type=jax.ShapeDtypeStruct((B,), jnp.float32),
        scratch_types=[
            pltpu.VMEM((nf * BPW,), jnp.int32),
            pltpu.VMEM((N * BPW,), jnp.float32),
            pltpu.VMEM((nf * BPW,), jnp.float32),
            pltpu.VMEM((N + 1, L), jnp.float32),
            pltpu.VMEM((BPW,), jnp.float32),
            pltpu.VMEM((BPW,), jnp.float32),
            pltpu.SemaphoreType.DMA,
            pltpu.SemaphoreType.DMA,
        ],
    )
    return f(s1d, c1d, n1d, wnb, prev)


def kernel(c_in, n_in, tables, W, b):
    wflat = W[:, 0]
    w2d = wflat[:F * D].reshape(F, D)
    wnb = jnp.broadcast_to(
        jnp.concatenate([wflat[F * D:], b])[:, None], (N + 1, L))
    c32 = c_in.astype(jnp.int32)
    n_i = lax.bitcast_convert_type(n_in, jnp.int32)
    c_flat, n1d_i = _detile_sc(c32, n_i)
    n_flat = lax.bitcast_convert_type(n1d_i, jnp.float32)

    t2 = jnp.transpose(tables, (0, 2, 1))
    scores = []
    f_lo = 0
    for nf in PHASES:
        scores.append(_score_tc(t2, w2d, nf, f_lo))
        f_lo += nf
    out = None
    f_lo = 0
    for i, nf in enumerate(PHASES):
        out = _gather_sc(scores[i], c_flat, n_flat, wnb,
                         n_flat if out is None else out, nf, f_lo, i == 0)
        f_lo += nf
    return out.reshape(B, 1)


# speedup vs baseline: 1.0428x; 1.0428x over previous
import functools

import jax
import jax.numpy as jnp
from jax import lax
from jax.experimental import pallas as pl
from jax.experimental.pallas import tpu as pltpu
from jax.experimental.pallas import tpu_sc as plsc

B = 16384
F = 26
V = 100000
D = 16
N = 13

NC = 2
NS = 16
L = 16
NW = NC * NS
BPW = B // NW
CH = 128

VP = 102400
PHASES = (17, 9)


def _score_body(f_lo, w_ref, t_ref, s_ref):
    fi = pl.program_id(0) + f_lo
    x = t_ref[0]
    wv = w_ref[pl.ds(fi, 1)]
    s_ref[...] = jnp.dot(wv, x)[0]


def _score_tc(t2, w2d, nf, f_lo):
    return pl.pallas_call(
        functools.partial(_score_body, f_lo),
        grid=(nf,),
        compiler_params=pltpu.CompilerParams(
            dimension_semantics=("parallel",)),
        in_specs=[
            pl.BlockSpec((F, D), lambda f: (0, 0)),
            pl.BlockSpec((1, D, VP), lambda f: (f + f_lo, 0, 0)),
        ],
        out_specs=pl.BlockSpec((VP,), lambda f: (f,)),
        out_shape=jax.ShapeDtypeStruct((nf * VP,), jnp.float32),
    )(w2d, t2)


def _detile_body(c_in, n_in, c1d, n1d, cw, cv, nw, nv, sem):
    wid = lax.axis_index("s") * NC + lax.axis_index("c")
    base = wid * BPW
    cps = []
    for f in range(F):
        cps.append(pltpu.async_copy(
            c_in.at[pl.ds(f, 1), pl.ds(base, BPW)], cw.at[pl.ds(f, 1)], sem))
    for n in range(N):
        cps.append(pltpu.async_copy(
            n_in.at[pl.ds(n, 1), pl.ds(base, BPW)], nw.at[pl.ds(n, 1)], sem))
    for cp in cps:
        cp.wait()
    for f in range(F):
        def kbody(k, carry, f=f):
            sl = pl.ds(k * L, L)
            cv[pl.ds(f * BPW + k * L, L)] = cw[f, sl]
            return carry
        lax.fori_loop(0, BPW // L, kbody, 0)
    for n in range(N):
        def kbody(k, carry, n=n):
            sl = pl.ds(k * L, L)
            nv[pl.ds(n * BPW + k * L, L)] = nw[n, sl]
            return carry
        lax.fori_loop(0, BPW // L, kbody, 0)
    cps = []
    for f in range(F):
        cps.append(pltpu.async_copy(cv.at[pl.ds(f * BPW, BPW)],
                                    c1d.at[pl.ds(f * B + base, BPW)], sem))
    for n in range(N):
        cps.append(pltpu.async_copy(nv.at[pl.ds(n * BPW, BPW)],
                                    n1d.at[pl.ds(n * B + base, BPW)], sem))
    for cp in cps:
        cp.wait()


def _detile_sc(c_in, n_in):
    mesh = plsc.VectorSubcoreMesh(core_axis_name="c", subcore_axis_name="s")
    f = pl.kernel(
        _detile_body,
        mesh=mesh,
        compiler_params=pltpu.CompilerParams(use_tc_tiling_on_sc=True),
        out_type=[
            jax.ShapeDtypeStruct((F * B,), jnp.int32),
            jax.ShapeDtypeStruct((N * B,), jnp.float32),
        ],
        scratch_types=[
            pltpu.VMEM((F, BPW), jnp.int32),
            pltpu.VMEM((F * BPW,), jnp.int32),
            pltpu.VMEM((N, BPW), jnp.float32),
            pltpu.VMEM((N * BPW,), jnp.float32),
            pltpu.SemaphoreType.DMA,
        ],
    )
    return f(c_in, n_in)


def _gather_body(nf, f_lo, with_nb, s1d, c1d, n1d, wnb, prev, out,
                 cbuf, nbuf, gbuf, wnbv, outv, pbuf, sem, gsem):
    nst = (nf * BPW) // CH
    wid = lax.axis_index("s") * NC + lax.axis_index("c")
    base = wid * BPW

    cps = [pltpu.async_copy(wnb, wnbv, sem)]
    for fl in range(nf):
        cps.append(pltpu.async_copy(
            c1d.at[pl.ds((f_lo + fl) * B + base, BPW)],
            cbuf.at[pl.ds(fl * BPW, BPW)], sem))
    if with_nb:
        for n in range(N):
            cps.append(pltpu.async_copy(n1d.at[pl.ds(n * B + base, BPW)],
                                        nbuf.at[pl.ds(n * BPW, BPW)], sem))
    else:
        cps.append(pltpu.async_copy(prev.at[pl.ds(base, BPW)], pbuf, sem))
    for cp in cps:
        cp.wait()

    for i in range(nst):
        fl = i // (BPW // CH)
        sl = pl.ds(i * CH, CH)
        pltpu.async_copy(s1d.at[pl.ds(fl * VP, VP)].at[cbuf.at[sl]],
                         gbuf.at[sl], gsem)

    pltpu.make_async_copy(s1d.at[pl.ds(0, nf * BPW)], gbuf, gsem).wait()

    wn_b = [wnbv[n] for n in range(N)]
    bias = wnbv[N]

    def gbody(g, carry):
        val = gbuf[pl.ds(g * L, L)]
        for fl in range(1, nf):
            val = val + gbuf[pl.ds(fl * BPW + g * L, L)]
        if with_nb:
            val = val + bias
            for n in range(N):
                val = val + nbuf[pl.ds(n * BPW + g * L, L)] * wn_b[n]
        else:
            val = val + pbuf[pl.ds(g * L, L)]
        outv[pl.ds(g * L, L)] = val
        return carry
    lax.fori_loop(0, BPW // L, gbody, 0)

    pltpu.sync_copy(outv, out.at[pl.ds(base, BPW)])


def _gather_sc(s1d, c1d, n1d, wnb, prev, nf, f_lo, with_nb):
    mesh = plsc.VectorSubcoreMesh(core_axis_name="c", subcore_axis_name="s")
    f = pl.kernel(
        functools.partial(_gather_body, nf, f_lo, with_nb),
        mesh=mesh,
        compiler_params=pltpu.CompilerParams(use_tc_tiling_on_sc=False),
        out_type=jax.ShapeDtypeStruct((B,), jnp.float32),
        scratch_types=[
            pltpu.VMEM((nf * BPW,), jnp.int32),
            pltpu.VMEM((N * BPW,), jnp.float32),
            pltpu.VMEM((nf * BPW,), jnp.float32),
            pltpu.VMEM((N + 1, L), jnp.float32),
            pltpu.VMEM((BPW,), jnp.float32),
            pltpu.VMEM((BPW,), jnp.float32),
            pltpu.SemaphoreType.DMA,
            pltpu.SemaphoreType.DMA,
        ],
    )
    return f(s1d, c1d, n1d, wnb, prev)


def kernel(c_in, n_in, tables, W, b):
    wflat = W[:, 0]
    w2d = wflat[:F * D].reshape(F, D)
    wnb = jnp.broadcast_to(
        jnp.concatenate([wflat[F * D:], b])[:, None], (N + 1, L))
    c32 = c_in.astype(jnp.int32)
    c_flat, n_flat = _detile_sc(c32, n_in)

    t2 = jnp.transpose(tables, (0, 2, 1))
    scores = []
    f_lo = 0
    for nf in PHASES:
        scores.append(_score_tc(t2, w2d, nf, f_lo))
        f_lo += nf
    out = None
    f_lo = 0
    for i, nf in enumerate(PHASES):
        out = _gather_sc(scores[i], c_flat, n_flat, wnb,
                         n_flat if out is None else out, nf, f_lo, i == 0)
        f_lo += nf
    return out.reshape(B, 1)
